# Initial kernel scaffold; baseline (speedup 1.0000x reference)
import jax
import jax.numpy as jnp
from jax.experimental import pallas as pl


def kernel(distance_matrix):
    raise NotImplementedError("write your pallas kernel here")


# speedup vs baseline: 5.7445x; 5.7445x over previous
import jax
import jax.numpy as jnp
from jax import lax
from jax.experimental import pallas as pl

_K = 16
_RADIUS = 0.8
_BINS = 50
_WIDTH = 0.2


def _select_kernel(d_ref, cols_ref, vals_ref):
    d = d_ref[...]
    r, n = d.shape
    colidx = lax.broadcasted_iota(jnp.int32, (r, n), 1)
    key = jnp.where(d <= _RADIUS, d, jnp.inf)
    alive = jnp.ones((r, n), dtype=jnp.bool_)
    cols_list = []
    vals_list = []
    for _ in range(_K):
        keyv = jnp.where(alive, key, jnp.inf)
        v = jnp.min(keyv, axis=1, keepdims=True)
        cand = jnp.where(alive & (keyv == v), colidx, n)
        idx = jnp.min(cand, axis=1, keepdims=True)
        hit = colidx == idx
        dval = jnp.sum(jnp.where(hit, d, 0.0), axis=1, keepdims=True)
        alive = jnp.logical_and(alive, jnp.logical_not(hit))
        cols_list.append(idx)
        vals_list.append(dval)
    cols_ref[...] = jnp.concatenate(cols_list, axis=1)
    vals_ref[...] = jnp.concatenate(vals_list, axis=1)


def _feature_kernel(v_ref, feat_ref):
    dflat = v_ref[...]
    centers = lax.broadcasted_iota(jnp.int32, (1, _BINS), 1).astype(jnp.float32) * (
        1.0 / (_BINS - 1))
    z = (dflat - centers) * (1.0 / _WIDTH)
    feat_ref[...] = jnp.exp(-0.5 * z * z)


def kernel(distance_matrix):
    n = distance_matrix.shape[0]
    r = min(256, n)
    grid = n // r
    cols, vals = pl.pallas_call(
        _select_kernel,
        grid=(grid,),
        in_specs=[pl.BlockSpec((r, n), lambda i: (i, 0))],
        out_specs=[
            pl.BlockSpec((r, _K), lambda i: (i, 0)),
            pl.BlockSpec((r, _K), lambda i: (i, 0)),
        ],
        out_shape=[
            jax.ShapeDtypeStruct((n, _K), jnp.int32),
            jax.ShapeDtypeStruct((n, _K), jnp.float32),
        ],
    )(distance_matrix)
    e = n * _K
    fb = min(8192, e)
    feats = pl.pallas_call(
        _feature_kernel,
        grid=(e // fb,),
        in_specs=[pl.BlockSpec((fb, 1), lambda i: (i, 0))],
        out_specs=pl.BlockSpec((fb, _BINS), lambda i: (i, 0)),
        out_shape=jax.ShapeDtypeStruct((e, _BINS), jnp.float32),
    )(vals.reshape(e, 1))
    rows = jnp.broadcast_to(jnp.arange(n, dtype=cols.dtype)[:, None], cols.shape)
    edge_index = jnp.stack([rows.reshape(-1), cols.reshape(-1)], axis=1)
    return edge_index, feats
